# Initial kernel scaffold; baseline (speedup 1.0000x reference)
import jax
import jax.numpy as jnp
from jax.experimental import pallas as pl


def kernel(x, edge_index, edge_attr, Wk, bk, root_kernel, conv_bias, gamma, beta, moving_mean, moving_var, Wd, bd):
    raise NotImplementedError("write your pallas kernel here")


# speedup vs baseline: 4.2817x; 4.2817x over previous
import functools

import jax
import jax.numpy as jnp
from jax import lax
from jax.experimental import pallas as pl
from jax.experimental.pallas import tpu as pltpu
from jax.experimental.pallas import tpu_sc as plsc

N_NODES = 10000
N_EDGES = 160000
F_IN = 128
F_OUT = 16
D_EDGE = 16
BN_EPS = 1e-3

NC = 2
NS = 16
NW = NC * NS
C = 128
NBLK = N_EDGES // C
BLK_PER_W = (NBLK + NW - 1) // NW
ROWS_PER_TILE = N_NODES // NS
D_Z = (D_EDGE + 1) * F_OUT


def _precompute_body(x_ref, wa_ref, rk_ref, z_ref, r_ref):
    x = x_ref[...]
    z_ref[...] = jnp.dot(x, wa_ref[...], preferred_element_type=jnp.float32)
    r_ref[...] = jnp.dot(x, rk_ref[...], preferred_element_type=jnp.float32)


def _edge_body(z_hbm, ea_hbm, src_hbm, dst_hbm, out_hbm,
               src_v, dst_v, zrows, ea_v, msgs, ztile, agg_sh, sem):
    cid = lax.axis_index("c")
    sid = lax.axis_index("s")
    wid = sid * NC + cid

    zero16 = jnp.zeros((16,), jnp.float32)

    def _zrow(i, carry):
        ztile[i, :] = zero16
        return carry

    lax.fori_loop(0, ROWS_PER_TILE, _zrow, 0)
    pltpu.sync_copy(ztile, agg_sh.at[pl.ds(sid * ROWS_PER_TILE, ROWS_PER_TILE)])
    plsc.subcore_barrier()

    def _chunk(i, carry):
        b = wid + i * NW

        @pl.when(b < NBLK)
        def _():
            base = b * C
            pltpu.sync_copy(src_hbm.at[pl.ds(base, C)], src_v)
            pltpu.sync_copy(dst_hbm.at[pl.ds(base, C)], dst_v)
            pltpu.sync_copy(ea_hbm.at[pl.ds(base, C)], ea_v)
            pltpu.async_copy(z_hbm.at[src_v], zrows, sem).wait()

            def _edge(e, ecarry):
                ea_row = ea_v[e, :]
                acc = zrows[e, pl.ds(D_EDGE * F_OUT, F_OUT)]
                for d in range(D_EDGE):
                    acc = acc + ea_row[d] * zrows[e, pl.ds(d * F_OUT, F_OUT)]
                msgs[e, :] = acc
                return ecarry

            lax.fori_loop(0, C, _edge, 0)
            pltpu.sync_copy(msgs, agg_sh.at[dst_v], add=True)

        return carry

    lax.fori_loop(0, BLK_PER_W, _chunk, 0)

    plsc.subcore_barrier()
    pltpu.sync_copy(
        agg_sh.at[pl.ds(sid * ROWS_PER_TILE, ROWS_PER_TILE)],
        out_hbm.at[cid, pl.ds(sid * ROWS_PER_TILE, ROWS_PER_TILE)],
    )


def _epilogue_body(p_ref, r_ref, cb_ref, gamma_ref, beta_ref, mean_ref,
                   var_ref, wd_ref, bd_ref, out_ref):
    agg = p_ref[0] + p_ref[1] + r_ref[...] + cb_ref[...]
    out = jnp.maximum(agg, 0.0)
    scale = gamma_ref[...] * lax.rsqrt(var_ref[...] + BN_EPS)
    out = (out - mean_ref[...]) * scale + beta_ref[...]
    pooled = jnp.max(out, axis=0, keepdims=True)
    out_ref[...] = (
        jnp.dot(pooled, wd_ref[...], preferred_element_type=jnp.float32)
        + bd_ref[...]
    )


def kernel(x, edge_index, edge_attr, Wk, bk, root_kernel, conv_bias,
           gamma, beta, moving_mean, moving_var, Wd, bd):
    src = edge_index[0]
    dst = edge_index[1]

    wk_r = jnp.transpose(Wk.reshape(D_EDGE, F_IN, F_OUT), (1, 0, 2))
    wa = jnp.concatenate(
        [wk_r.reshape(F_IN, D_EDGE * F_OUT), bk.reshape(F_IN, F_OUT)], axis=1)

    z, r = pl.pallas_call(
        _precompute_body,
        out_shape=[
            jax.ShapeDtypeStruct((N_NODES, D_Z), jnp.float32),
            jax.ShapeDtypeStruct((N_NODES, F_OUT), jnp.float32),
        ],
    )(x, wa, root_kernel)

    mesh = plsc.VectorSubcoreMesh(core_axis_name="c", subcore_axis_name="s")
    edge_fn = functools.partial(
        pl.kernel,
        out_type=jax.ShapeDtypeStruct((NC, N_NODES, F_OUT), jnp.float32),
        mesh=mesh,
        compiler_params=pltpu.CompilerParams(use_tc_tiling_on_sc=False),
        scratch_types=[
            pltpu.VMEM((C,), jnp.int32),
            pltpu.VMEM((C,), jnp.int32),
            pltpu.VMEM((C, D_Z), jnp.float32),
            pltpu.VMEM((C, D_EDGE), jnp.float32),
            pltpu.VMEM((C, F_OUT), jnp.float32),
            pltpu.VMEM((ROWS_PER_TILE, F_OUT), jnp.float32),
            pltpu.VMEM_SHARED((N_NODES, F_OUT), jnp.float32),
            pltpu.SemaphoreType.DMA,
        ],
    )(_edge_body)
    partials = edge_fn(z, edge_attr, src, dst)

    logits = pl.pallas_call(
        _epilogue_body,
        out_shape=jax.ShapeDtypeStruct((1, 3), jnp.float32),
    )(
        partials, r,
        conv_bias.reshape(1, F_OUT),
        gamma.reshape(1, F_OUT),
        beta.reshape(1, F_OUT),
        moving_mean.reshape(1, F_OUT),
        moving_var.reshape(1, F_OUT),
        Wd,
        bd.reshape(1, 3),
    )
    return logits


# speedup vs baseline: 4.5529x; 1.0633x over previous
import functools

import jax
import jax.numpy as jnp
from jax import lax
from jax.experimental import pallas as pl
from jax.experimental.pallas import tpu as pltpu
from jax.experimental.pallas import tpu_sc as plsc

N_NODES = 10000
N_EDGES = 160000
F_IN = 128
F_OUT = 16
D_EDGE = 16
BN_EPS = 1e-3

NC = 2
NS = 16
NW = NC * NS
C = 128
NBLK = N_EDGES // C
BLK_PER_W = (NBLK + NW - 1) // NW
ROWS_PER_TILE = N_NODES // NS
D_Z = (D_EDGE + 1) * F_OUT


def _precompute_body(x_ref, wa_ref, rk_ref, z_ref, r_ref):
    x = x_ref[...]
    z_ref[...] = jnp.dot(x, wa_ref[...], preferred_element_type=jnp.float32)
    r_ref[...] = jnp.dot(x, rk_ref[...], preferred_element_type=jnp.float32)


def _edge_body(z_hbm, ei_hbm, ea_hbm, out_hbm,
               src_v, dst_v, zrows, ea_v, msgs, ztile, agg_sh, sem):
    cid = lax.axis_index("c")
    sid = lax.axis_index("s")
    wid = sid * NC + cid

    zero16 = jnp.zeros((16,), jnp.float32)

    def _zrow(i, carry):
        ztile[i, :] = zero16
        return carry

    lax.fori_loop(0, ROWS_PER_TILE, _zrow, 0)
    pltpu.sync_copy(ztile, agg_sh.at[pl.ds(sid * ROWS_PER_TILE, ROWS_PER_TILE)])
    plsc.subcore_barrier()

    def _chunk(i, carry):
        b = wid + i * NW

        @pl.when(b < NBLK)
        def _():
            base = b * C
            pltpu.sync_copy(ei_hbm.at[0, pl.ds(base, C)], src_v)
            pltpu.sync_copy(ei_hbm.at[1, pl.ds(base, C)], dst_v)
            pltpu.sync_copy(ea_hbm.at[pl.ds(base, C)], ea_v)
            pltpu.async_copy(z_hbm.at[src_v], zrows, sem).wait()

            def _edge(e, ecarry):
                ea_row = ea_v[e, :]
                t = [ea_row[d] * zrows[e, pl.ds(d * F_OUT, F_OUT)]
                     for d in range(D_EDGE)]
                t.append(zrows[e, pl.ds(D_EDGE * F_OUT, F_OUT)])
                while len(t) > 1:
                    t = [t[i] + t[i + 1] for i in range(0, len(t) - 1, 2)] + (
                        [t[-1]] if len(t) % 2 else [])
                msgs[e, :] = t[0]
                return ecarry

            lax.fori_loop(0, C, _edge, 0, unroll=4)
            pltpu.sync_copy(msgs, agg_sh.at[dst_v], add=True)

        return carry

    lax.fori_loop(0, BLK_PER_W, _chunk, 0)

    plsc.subcore_barrier()
    pltpu.sync_copy(
        agg_sh.at[pl.ds(sid * ROWS_PER_TILE, ROWS_PER_TILE)],
        out_hbm.at[cid, pl.ds(sid * ROWS_PER_TILE, ROWS_PER_TILE)],
    )


def _epilogue_body(p_ref, r_ref, cb_ref, gamma_ref, beta_ref, mean_ref,
                   var_ref, wd_ref, bd_ref, out_ref):
    agg = p_ref[0] + p_ref[1] + r_ref[...] + cb_ref[...]
    out = jnp.maximum(agg, 0.0)
    scale = gamma_ref[...] * lax.rsqrt(var_ref[...] + BN_EPS)
    out = (out - mean_ref[...]) * scale + beta_ref[...]
    pooled = jnp.max(out, axis=0, keepdims=True)
    out_ref[...] = (
        jnp.dot(pooled, wd_ref[...], preferred_element_type=jnp.float32)
        + bd_ref[...]
    )


def kernel(x, edge_index, edge_attr, Wk, bk, root_kernel, conv_bias,
           gamma, beta, moving_mean, moving_var, Wd, bd):
    wk_r = jnp.transpose(Wk.reshape(D_EDGE, F_IN, F_OUT), (1, 0, 2))
    wa = jnp.concatenate(
        [wk_r.reshape(F_IN, D_EDGE * F_OUT), bk.reshape(F_IN, F_OUT)], axis=1)

    z, r = pl.pallas_call(
        _precompute_body,
        out_shape=[
            jax.ShapeDtypeStruct((N_NODES, D_Z), jnp.float32),
            jax.ShapeDtypeStruct((N_NODES, F_OUT), jnp.float32),
        ],
    )(x, wa, root_kernel)

    mesh = plsc.VectorSubcoreMesh(core_axis_name="c", subcore_axis_name="s")
    edge_fn = functools.partial(
        pl.kernel,
        out_type=jax.ShapeDtypeStruct((NC, N_NODES, F_OUT), jnp.float32),
        mesh=mesh,
        compiler_params=pltpu.CompilerParams(use_tc_tiling_on_sc=False),
        scratch_types=[
            pltpu.VMEM((C,), jnp.int32),
            pltpu.VMEM((C,), jnp.int32),
            pltpu.VMEM((C, D_Z), jnp.float32),
            pltpu.VMEM((C, D_EDGE), jnp.float32),
            pltpu.VMEM((C, F_OUT), jnp.float32),
            pltpu.VMEM((ROWS_PER_TILE, F_OUT), jnp.float32),
            pltpu.VMEM_SHARED((N_NODES, F_OUT), jnp.float32),
            pltpu.SemaphoreType.DMA,
        ],
    )(_edge_body)
    partials = edge_fn(z, edge_index, edge_attr)

    logits = pl.pallas_call(
        _epilogue_body,
        out_shape=jax.ShapeDtypeStruct((1, 3), jnp.float32),
    )(
        partials, r,
        conv_bias.reshape(1, F_OUT),
        gamma.reshape(1, F_OUT),
        beta.reshape(1, F_OUT),
        moving_mean.reshape(1, F_OUT),
        moving_var.reshape(1, F_OUT),
        Wd,
        bd.reshape(1, 3),
    )
    return logits


# speedup vs baseline: 7.1914x; 1.5795x over previous
import functools

import jax
import jax.numpy as jnp
from jax import lax
from jax.experimental import pallas as pl
from jax.experimental.pallas import tpu as pltpu
from jax.experimental.pallas import tpu_sc as plsc

N_NODES = 10000
N_EDGES = 160000
F_IN = 128
F_OUT = 16
D_EDGE = 16
BN_EPS = 1e-3

NC = 2
NS = 16
NW = NC * NS
C = 128
NBLK = N_EDGES // C
BASE_BLK = NBLK // NW
EXTRA_W = NBLK - BASE_BLK * NW
MAX_BLK = BASE_BLK + 1
HALF_IT = (MAX_BLK + 1) // 2
ROWS_PER_TILE = N_NODES // NS
D_Z = (D_EDGE + 1) * F_OUT


def _precompute_body(x_ref, wa_ref, rk_ref, z_ref, r_ref):
    x = x_ref[...]
    z_ref[...] = jnp.dot(x, wa_ref[...], preferred_element_type=jnp.float32)
    r_ref[...] = jnp.dot(x, rk_ref[...], preferred_element_type=jnp.float32)


def _edge_body(z_hbm, ei_hbm, ea_hbm, out_hbm,
               srcall, dstall, zbuf, eabuf, msgs, ztile, agg_sh,
               gsem0, gsem1, easem0, easem1):
    cid = lax.axis_index("c")
    sid = lax.axis_index("s")
    wid = sid * NC + cid
    nblk = BASE_BLK + jnp.where(wid < EXTRA_W, 1, 0)
    start = BASE_BLK * wid + jnp.minimum(wid, EXTRA_W)

    gsems = (gsem0, gsem1)
    easems = (easem0, easem1)

    pltpu.sync_copy(ei_hbm.at[0, pl.ds(start, BASE_BLK)],
                    srcall.at[pl.ds(0, BASE_BLK)])
    pltpu.sync_copy(ei_hbm.at[1, pl.ds(start, BASE_BLK)],
                    dstall.at[pl.ds(0, BASE_BLK)])

    @pl.when(wid < EXTRA_W)
    def _():
        pltpu.sync_copy(ei_hbm.at[0, pl.ds(start + BASE_BLK, 1)],
                        srcall.at[pl.ds(BASE_BLK, 1)])
        pltpu.sync_copy(ei_hbm.at[1, pl.ds(start + BASE_BLK, 1)],
                        dstall.at[pl.ds(BASE_BLK, 1)])

    zero16 = jnp.zeros((16,), jnp.float32)

    def _zrow(i, carry):
        ztile[i, :] = zero16
        return carry

    lax.fori_loop(0, ROWS_PER_TILE, _zrow, 0)
    pltpu.sync_copy(ztile, agg_sh.at[pl.ds(sid * ROWS_PER_TILE, ROWS_PER_TILE)])
    plsc.subcore_barrier()

    def _fire(i, slot):
        pltpu.async_copy(z_hbm.at[srcall.at[i]], zbuf.at[slot], gsems[slot])
        pltpu.async_copy(ea_hbm.at[start + i], eabuf.at[slot], easems[slot])

    def _drain(i, slot):
        pltpu.make_async_copy(z_hbm.at[srcall.at[i]], zbuf.at[slot],
                              gsems[slot]).wait()
        pltpu.make_async_copy(ea_hbm.at[start + i], eabuf.at[slot],
                              easems[slot]).wait()

    def _consume(i, slot):
        _drain(i, slot)

        def _edge(e, ecarry):
            ea_row = eabuf[slot, e, :]
            t = [ea_row[d] * zbuf[slot, e, pl.ds(d * F_OUT, F_OUT)]
                 for d in range(D_EDGE)]
            t.append(zbuf[slot, e, pl.ds(D_EDGE * F_OUT, F_OUT)])
            while len(t) > 1:
                t = [t[i2] + t[i2 + 1] for i2 in range(0, len(t) - 1, 2)] + (
                    [t[-1]] if len(t) % 2 else [])
            msgs[e, :] = t[0]
            return ecarry

        lax.fori_loop(0, C, _edge, 0, unroll=4)
        pltpu.sync_copy(msgs, agg_sh.at[dstall.at[i]], add=True)

    @pl.when(0 < nblk)
    def _():
        _fire(0, 0)

    def _pipe(j, carry):
        b0 = 2 * j
        b1 = 2 * j + 1

        @pl.when(b1 < nblk)
        def _():
            _fire(b1, 1)

        @pl.when(b0 < nblk)
        def _():
            _consume(b0, 0)

        @pl.when(b0 + 2 < nblk)
        def _():
            _fire(b0 + 2, 0)

        @pl.when(b1 < nblk)
        def _():
            _consume(b1, 1)

        return carry

    lax.fori_loop(0, HALF_IT, _pipe, 0)

    plsc.subcore_barrier()
    pltpu.sync_copy(
        agg_sh.at[pl.ds(sid * ROWS_PER_TILE, ROWS_PER_TILE)],
        out_hbm.at[cid, pl.ds(sid * ROWS_PER_TILE, ROWS_PER_TILE)],
    )


def _epilogue_body(p_ref, r_ref, cb_ref, gamma_ref, beta_ref, mean_ref,
                   var_ref, wd_ref, bd_ref, out_ref):
    agg = p_ref[0] + p_ref[1] + r_ref[...] + cb_ref[...]
    out = jnp.maximum(agg, 0.0)
    scale = gamma_ref[...] * lax.rsqrt(var_ref[...] + BN_EPS)
    out = (out - mean_ref[...]) * scale + beta_ref[...]
    pooled = jnp.max(out, axis=0, keepdims=True)
    out_ref[...] = (
        jnp.dot(pooled, wd_ref[...], preferred_element_type=jnp.float32)
        + bd_ref[...]
    )


def kernel(x, edge_index, edge_attr, Wk, bk, root_kernel, conv_bias,
           gamma, beta, moving_mean, moving_var, Wd, bd):
    wk_r = jnp.transpose(Wk.reshape(D_EDGE, F_IN, F_OUT), (1, 0, 2))
    wa = jnp.concatenate(
        [wk_r.reshape(F_IN, D_EDGE * F_OUT), bk.reshape(F_IN, F_OUT)], axis=1)

    z, r = pl.pallas_call(
        _precompute_body,
        out_shape=[
            jax.ShapeDtypeStruct((N_NODES, D_Z), jnp.float32),
            jax.ShapeDtypeStruct((N_NODES, F_OUT), jnp.float32),
        ],
    )(x, wa, root_kernel)

    ei3 = edge_index.reshape(2, NBLK, C)
    ea3 = edge_attr.reshape(NBLK, C, D_EDGE)

    mesh = plsc.VectorSubcoreMesh(core_axis_name="c", subcore_axis_name="s")
    edge_fn = functools.partial(
        pl.kernel,
        out_type=jax.ShapeDtypeStruct((NC, N_NODES, F_OUT), jnp.float32),
        mesh=mesh,
        compiler_params=pltpu.CompilerParams(use_tc_tiling_on_sc=False),
        scratch_types=[
            pltpu.VMEM((MAX_BLK, C), jnp.int32),
            pltpu.VMEM((MAX_BLK, C), jnp.int32),
            pltpu.VMEM((2, C, D_Z), jnp.float32),
            pltpu.VMEM((2, C, D_EDGE), jnp.float32),
            pltpu.VMEM((C, F_OUT), jnp.float32),
            pltpu.VMEM((ROWS_PER_TILE, F_OUT), jnp.float32),
            pltpu.VMEM_SHARED((N_NODES, F_OUT), jnp.float32),
            pltpu.SemaphoreType.DMA,
            pltpu.SemaphoreType.DMA,
            pltpu.SemaphoreType.DMA,
            pltpu.SemaphoreType.DMA,
        ],
    )(_edge_body)
    partials = edge_fn(z, ei3, ea3)

    logits = pl.pallas_call(
        _epilogue_body,
        out_shape=jax.ShapeDtypeStruct((1, 3), jnp.float32),
    )(
        partials, r,
        conv_bias.reshape(1, F_OUT),
        gamma.reshape(1, F_OUT),
        beta.reshape(1, F_OUT),
        moving_mean.reshape(1, F_OUT),
        moving_var.reshape(1, F_OUT),
        Wd,
        bd.reshape(1, 3),
    )
    return logits
